# Initial kernel scaffold; baseline (speedup 1.0000x reference)
import jax
import jax.numpy as jnp
from jax.experimental import pallas as pl


def kernel(fe, segment_ids, W, b):
    raise NotImplementedError("write your pallas kernel here")


# speedup vs baseline: 12.1549x; 12.1549x over previous
import functools

import jax
import jax.numpy as jnp
from jax import lax
from jax.experimental import pallas as pl
from jax.experimental.pallas import tpu as pltpu
from jax.experimental.pallas import tpu_sc as plsc

E = 3_200_000
DE = 16
DG = 128
NG = 256

_CHUNK = 2000


def _seg_reduce_sc(fe_flat, seg_ids):
    info = plsc.get_sparse_core_info()
    nc, ns = info.num_cores, info.num_subcores
    nw = nc * ns
    per_shard = E // nw
    n_chunks = per_shard // _CHUNK
    mesh = plsc.VectorSubcoreMesh(core_axis_name="c", subcore_axis_name="s")

    neg_inf = jnp.float32(-jnp.inf)
    pos_inf = jnp.float32(jnp.inf)

    @functools.partial(
        pl.kernel,
        mesh=mesh,
        out_type=[
            jax.ShapeDtypeStruct((nw, NG * DE), jnp.float32),
            jax.ShapeDtypeStruct((nw, NG * DE), jnp.float32),
            jax.ShapeDtypeStruct((nw, NG * DE), jnp.float32),
            jax.ShapeDtypeStruct((nw, NG * DE), jnp.float32),
        ],
        scratch_types=[
            pltpu.VMEM((_CHUNK * DE,), jnp.float32),
            pltpu.VMEM((_CHUNK,), jnp.int32),
            pltpu.VMEM((NG * DE,), jnp.float32),
            pltpu.VMEM((NG * DE,), jnp.float32),
            pltpu.VMEM((NG * DE,), jnp.float32),
            pltpu.VMEM((NG * DE,), jnp.float32),
        ],
    )
    def k(fe_hbm, ids_hbm, o_sum, o_min, o_max, o_cnt,
          fe_v, ids_v, sum_t, min_t, max_t, cnt_t):
        wid = lax.axis_index("c") * ns + lax.axis_index("s")
        base = wid * per_shard

        def init_row(j, _):
            sum_t[pl.ds(j * DE, DE)] = jnp.zeros((DE,), jnp.float32)
            min_t[pl.ds(j * DE, DE)] = jnp.full((DE,), pos_inf, jnp.float32)
            max_t[pl.ds(j * DE, DE)] = jnp.full((DE,), neg_inf, jnp.float32)
            cnt_t[pl.ds(j * DE, DE)] = jnp.zeros((DE,), jnp.float32)
            return 0

        lax.fori_loop(0, NG, init_row, 0)

        def flush(cur_id, cnt, s, mn, mx):
            @pl.when(cur_id >= 0)
            def _():
                off = cur_id * DE
                sum_t[pl.ds(off, DE)] = sum_t[pl.ds(off, DE)] + s
                min_t[pl.ds(off, DE)] = jnp.minimum(min_t[pl.ds(off, DE)], mn)
                max_t[pl.ds(off, DE)] = jnp.maximum(max_t[pl.ds(off, DE)], mx)
                cnt_t[pl.ds(off, DE)] = cnt_t[pl.ds(off, DE)] + jnp.full(
                    (DE,), cnt)

        def chunk_body(c, carry):
            start = base + c * _CHUNK
            pltpu.sync_copy(fe_hbm.at[pl.ds(start * DE, _CHUNK * DE)], fe_v)
            pltpu.sync_copy(ids_hbm.at[pl.ds(start, _CHUNK)], ids_v)

            def group_body(g, gcarry):
                idvec = ids_v[pl.ds(g * 16, 16)]
                rowbase = g * 16
                carry = gcarry
                for l in range(16):
                    cur_id, cnt, s, mn, mx = carry
                    eid = idvec[l]
                    row = fe_v[pl.ds((rowbase + l) * DE, DE)]
                    is_new = eid != cur_id

                    @pl.when(is_new)
                    def _(cur_id=cur_id, cnt=cnt, s=s, mn=mn, mx=mx):
                        flush(cur_id, cnt, s, mn, mx)

                    cur_id = lax.select(is_new, eid, cur_id)
                    cnt = lax.select(is_new, jnp.float32(1.0), cnt + 1.0)
                    s = jnp.where(is_new, row, s + row)
                    mn = jnp.where(is_new, row, jnp.minimum(mn, row))
                    mx = jnp.where(is_new, row, jnp.maximum(mx, row))
                    carry = (cur_id, cnt, s, mn, mx)
                return carry

            return lax.fori_loop(0, _CHUNK // 16, group_body, carry)

        zero_v = jnp.zeros((DE,), jnp.float32)
        carry0 = (jnp.int32(-1), jnp.float32(0.0), zero_v, zero_v, zero_v)
        cur_id, cnt, s, mn, mx = lax.fori_loop(0, n_chunks, chunk_body, carry0)
        flush(cur_id, cnt, s, mn, mx)

        pltpu.sync_copy(sum_t, o_sum.at[wid])
        pltpu.sync_copy(min_t, o_min.at[wid])
        pltpu.sync_copy(max_t, o_max.at[wid])
        pltpu.sync_copy(cnt_t, o_cnt.at[wid])

    return k(fe_flat, seg_ids)


def _merge_tc_body(ps_ref, pm_ref, px_ref, pc_ref, wt_ref, b_ref, out_ref):
    s = jnp.sum(ps_ref[...], axis=0)
    mn = jnp.min(pm_ref[...], axis=0)
    mx = jnp.max(px_ref[...], axis=0)
    c = jnp.sum(pc_ref[...], axis=0)
    mean = s / jnp.maximum(c, 1.0)
    z = jnp.concatenate([mean, mn, mx], axis=1)
    out_ref[...] = (
        jnp.dot(z, wt_ref[...], preferred_element_type=jnp.float32)
        + b_ref[...]
    )


def _merge_tc(ps, pm, px, pc, wt, b2):
    return pl.pallas_call(
        _merge_tc_body,
        out_shape=jax.ShapeDtypeStruct((NG, DG), jnp.float32),
    )(ps, pm, px, pc, wt, b2)


@jax.jit
def kernel(fe, segment_ids, W, b):
    ids = segment_ids.astype(jnp.int32)
    ps, pm, px, pc = _seg_reduce_sc(fe.reshape(-1), ids)
    nw = ps.shape[0]
    return _merge_tc(
        ps.reshape(nw, NG, DE), pm.reshape(nw, NG, DE),
        px.reshape(nw, NG, DE), pc.reshape(nw, NG, DE), W.T, b[None, :])


# speedup vs baseline: 16.8233x; 1.3841x over previous
import functools

import jax
import jax.numpy as jnp
from jax import lax
from jax.experimental import pallas as pl
from jax.experimental.pallas import tpu as pltpu
from jax.experimental.pallas import tpu_sc as plsc

E = 3_200_000
DE = 16
DG = 128
NG = 256

_CHUNK = 2000
_GROUPS = _CHUNK // 16


def _tree(op, vals):
    vals = list(vals)
    while len(vals) > 1:
        nxt = [op(vals[2 * i], vals[2 * i + 1]) for i in range(len(vals) // 2)]
        if len(vals) % 2:
            nxt.append(vals[-1])
        vals = nxt
    return vals[0]


def _seg_reduce_sc(fe_flat, seg_ids):
    info = plsc.get_sparse_core_info()
    nc, ns = info.num_cores, info.num_subcores
    nw = nc * ns
    per_shard = E // nw
    n_chunks = per_shard // _CHUNK
    mesh = plsc.VectorSubcoreMesh(core_axis_name="c", subcore_axis_name="s")

    neg_inf = jnp.float32(-jnp.inf)
    pos_inf = jnp.float32(jnp.inf)

    @functools.partial(
        pl.kernel,
        mesh=mesh,
        out_type=[
            jax.ShapeDtypeStruct((nw, NG * DE), jnp.float32),
            jax.ShapeDtypeStruct((nw, NG * DE), jnp.float32),
            jax.ShapeDtypeStruct((nw, NG * DE), jnp.float32),
            jax.ShapeDtypeStruct((nw, NG * DE), jnp.float32),
        ],
        scratch_types=[
            pltpu.VMEM((_CHUNK * DE,), jnp.float32),
            pltpu.VMEM((_CHUNK * DE,), jnp.float32),
            pltpu.VMEM((_CHUNK,), jnp.int32),
            pltpu.VMEM((_CHUNK,), jnp.int32),
            pltpu.VMEM((NG * DE,), jnp.float32),
            pltpu.VMEM((NG * DE,), jnp.float32),
            pltpu.VMEM((NG * DE,), jnp.float32),
            pltpu.VMEM((NG * DE,), jnp.float32),
            pltpu.VMEM((DE,), jnp.float32),
            pltpu.VMEM((DE,), jnp.float32),
            pltpu.VMEM((DE,), jnp.float32),
            pltpu.SMEM((1,), jnp.int32),
            pltpu.SMEM((1,), jnp.float32),
            pltpu.SemaphoreType.DMA,
            pltpu.SemaphoreType.DMA,
            pltpu.SemaphoreType.DMA,
            pltpu.SemaphoreType.DMA,
        ],
    )
    def k(fe_hbm, ids_hbm, o_sum, o_min, o_max, o_cnt,
          fe_v0, fe_v1, ids_v0, ids_v1, sum_t, min_t, max_t, cnt_t,
          acc_s, acc_mn, acc_mx, cur_ref, cnt_ref,
          fsem0, fsem1, isem0, isem1):
        wid = lax.axis_index("c") * ns + lax.axis_index("s")
        base = wid * per_shard
        fe_bufs = (fe_v0, fe_v1)
        ids_bufs = (ids_v0, ids_v1)
        fsems = (fsem0, fsem1)
        isems = (isem0, isem1)

        def fe_dma(c, b):
            start = base + c * _CHUNK
            return pltpu.make_async_copy(
                fe_hbm.at[pl.ds(start * DE, _CHUNK * DE)], fe_bufs[b],
                fsems[b])

        def ids_dma(c, b):
            start = base + c * _CHUNK
            return pltpu.make_async_copy(
                ids_hbm.at[pl.ds(start, _CHUNK)], ids_bufs[b], isems[b])

        def init_row(j, _):
            sum_t[pl.ds(j * DE, DE)] = jnp.zeros((DE,), jnp.float32)
            min_t[pl.ds(j * DE, DE)] = jnp.full((DE,), pos_inf, jnp.float32)
            max_t[pl.ds(j * DE, DE)] = jnp.full((DE,), neg_inf, jnp.float32)
            cnt_t[pl.ds(j * DE, DE)] = jnp.zeros((DE,), jnp.float32)
            return 0

        lax.fori_loop(0, NG, init_row, 0)
        cur_ref[0] = jnp.int32(-1)
        cnt_ref[0] = jnp.float32(0.0)
        acc_s[...] = jnp.zeros((DE,), jnp.float32)
        acc_mn[...] = jnp.full((DE,), pos_inf, jnp.float32)
        acc_mx[...] = jnp.full((DE,), neg_inf, jnp.float32)

        def flush():
            cur = cur_ref[0]

            @pl.when(cur >= 0)
            def _():
                off = cur * DE
                sum_t[pl.ds(off, DE)] = sum_t[pl.ds(off, DE)] + acc_s[...]
                min_t[pl.ds(off, DE)] = jnp.minimum(
                    min_t[pl.ds(off, DE)], acc_mn[...])
                max_t[pl.ds(off, DE)] = jnp.maximum(
                    max_t[pl.ds(off, DE)], acc_mx[...])
                cnt_t[pl.ds(off, DE)] = cnt_t[pl.ds(off, DE)] + jnp.full(
                    (DE,), cnt_ref[0])

        def process_chunk(fe_b, ids_b):
            def group_body(g, _):
                idvec = ids_b[pl.ds(g * 16, 16)]
                gbase = g * (16 * DE)
                cur = cur_ref[0]
                fast = jnp.logical_and(idvec[0] == cur, idvec[15] == cur)

                @pl.when(fast)
                def _():
                    rows = [fe_b[pl.ds(gbase + l * DE, DE)]
                            for l in range(16)]
                    acc_s[...] = acc_s[...] + _tree(jnp.add, rows)
                    acc_mn[...] = jnp.minimum(
                        acc_mn[...], _tree(jnp.minimum, rows))
                    acc_mx[...] = jnp.maximum(
                        acc_mx[...], _tree(jnp.maximum, rows))
                    cnt_ref[0] = cnt_ref[0] + 16.0

                @pl.when(jnp.logical_not(fast))
                def _():
                    for l in range(16):
                        eid = idvec[l]

                        @pl.when(eid != cur_ref[0])
                        def _(eid=eid):
                            flush()
                            cur_ref[0] = eid
                            cnt_ref[0] = jnp.float32(0.0)
                            acc_s[...] = jnp.zeros((DE,), jnp.float32)
                            acc_mn[...] = jnp.full(
                                (DE,), pos_inf, jnp.float32)
                            acc_mx[...] = jnp.full(
                                (DE,), neg_inf, jnp.float32)

                        row = fe_b[pl.ds(gbase + l * DE, DE)]
                        acc_s[...] = acc_s[...] + row
                        acc_mn[...] = jnp.minimum(acc_mn[...], row)
                        acc_mx[...] = jnp.maximum(acc_mx[...], row)
                        cnt_ref[0] = cnt_ref[0] + 1.0

                return 0

            lax.fori_loop(0, _GROUPS, group_body, 0)

        fe_dma(0, 0).start()
        ids_dma(0, 0).start()

        def pair_body(i, _):
            for b in range(2):
                c = i * 2 + b

                @pl.when(c + 1 < n_chunks)
                def _(c=c, b=b):
                    fe_dma(c + 1, 1 - b).start()
                    ids_dma(c + 1, 1 - b).start()

                fe_dma(c, b).wait()
                ids_dma(c, b).wait()
                process_chunk(fe_bufs[b], ids_bufs[b])
            return 0

        lax.fori_loop(0, n_chunks // 2, pair_body, 0)
        flush()

        pltpu.sync_copy(sum_t, o_sum.at[wid])
        pltpu.sync_copy(min_t, o_min.at[wid])
        pltpu.sync_copy(max_t, o_max.at[wid])
        pltpu.sync_copy(cnt_t, o_cnt.at[wid])

    return k(fe_flat, seg_ids)


def _merge_tc_body(ps_ref, pm_ref, px_ref, pc_ref, wt_ref, b_ref, out_ref):
    s = jnp.sum(ps_ref[...], axis=0)
    mn = jnp.min(pm_ref[...], axis=0)
    mx = jnp.max(px_ref[...], axis=0)
    c = jnp.sum(pc_ref[...], axis=0)
    mean = s / jnp.maximum(c, 1.0)
    z = jnp.concatenate([mean, mn, mx], axis=1)
    out_ref[...] = (
        jnp.dot(z, wt_ref[...], preferred_element_type=jnp.float32)
        + b_ref[...]
    )


def _merge_tc(ps, pm, px, pc, wt, b2):
    return pl.pallas_call(
        _merge_tc_body,
        out_shape=jax.ShapeDtypeStruct((NG, DG), jnp.float32),
    )(ps, pm, px, pc, wt, b2)


@jax.jit
def kernel(fe, segment_ids, W, b):
    ids = segment_ids.astype(jnp.int32)
    ps, pm, px, pc = _seg_reduce_sc(fe.reshape(-1), ids)
    nw = ps.shape[0]
    return _merge_tc(
        ps.reshape(nw, NG, DE), pm.reshape(nw, NG, DE),
        px.reshape(nw, NG, DE), pc.reshape(nw, NG, DE), W.T, b[None, :])
